# baseline (speedup 1.0000x reference)
import jax
import jax.numpy as jnp
from jax.experimental import pallas as pl

_B, _C, _T, _H, _W = 32, 384, 16, 14, 14
_C2 = 768
_KSEL = 8
_NS = 1000
_SIGMA = 0.05
_HW = _H * _W
_CB = 128
_INTERPRET = False


def _ind_krn(pt_ref, ind_ref):
    p = pt_ref[...].reshape(_T, _NS)
    pt = p[:, None, :]
    pu = p[None, :, :]
    iu = jax.lax.broadcasted_iota(jnp.int32, (_T, _T, _NS), 1)
    it = jax.lax.broadcasted_iota(jnp.int32, (_T, _T, _NS), 0)
    tie = ((pu == pt) & (iu < it)).astype(jnp.float32)
    rank_d = jnp.sum((pu > pt).astype(jnp.float32) + tie, axis=1)
    rank_a = jnp.sum((pu < pt).astype(jnp.float32) + tie, axis=1)
    sel = (rank_d < float(_KSEL)).astype(jnp.float32)
    bsel = (rank_a < float(_T - _KSEL)).astype(jnp.float32)
    mu_ = jax.lax.broadcasted_iota(jnp.int32, (_T, _T), 0)
    mt_ = jax.lax.broadcasted_iota(jnp.int32, (_T, _T), 1)
    m_lt = (mu_ > mt_).astype(jnp.float32)
    ec_s = jax.lax.dot_general(m_lt, sel, (((1,), (0,)), ((), ())),
                               preferred_element_type=jnp.float32)
    ec_b = jax.lax.dot_general(m_lt, bsel, (((1,), (0,)), ((), ())),
                               preferred_element_type=jnp.float32)
    cols = []
    for j in range(_KSEL):
        cols.append(jnp.sum(sel * (ec_s == float(j)), axis=1, keepdims=True))
    for j in range(_T - _KSEL):
        cols.append(jnp.sum(bsel * (ec_b == float(j)), axis=1, keepdims=True))
    ind = jnp.transpose(jnp.concatenate(cols, axis=1)) * (1.0 / _NS)
    ind_ref[...] = ind.reshape(1, _T, _T)


def _apply_krn(x_ref, ind_ref, tk_ref, bk_ref):
    x = x_ref[0]
    iv = ind_ref[0]
    for j in range(_T):
        acc = x[:, 0:1, :] * iv[j, 0]
        for t in range(1, _T):
            acc = acc + x[:, t:t + 1, :] * iv[j, t]
        if j < _KSEL:
            tk_ref[0, :, j:j + 1, :] = acc
        else:
            bk_ref[0, :, j - _KSEL:j - _KSEL + 1, :] = acc


def kernel(frames, conv_w, conv_b, bn_g, bn_b, ln_g, ln_b,
           w1, b1, w2, b2, w3, b3, w4, b4):
    f32 = jnp.float32

    x = jax.lax.stop_gradient(frames)
    y = jax.lax.conv_general_dilated(x, conv_w, window_strides=(1, 2, 2),
                                     padding='VALID',
                                     dimension_numbers=('NCDHW', 'OIDHW', 'NCDHW'))
    y = y + conv_b[None, :, None, None, None]
    mu = jnp.mean(y, axis=(0, 2, 3, 4), keepdims=True)
    var = jnp.var(y, axis=(0, 2, 3, 4), keepdims=True)
    y = (y - mu) / jnp.sqrt(var + 1e-5)
    y = y * bn_g[None, :, None, None, None] + bn_b[None, :, None, None, None]
    B, C, T, h, w = y.shape
    y = jnp.transpose(y, (0, 2, 3, 4, 1)).reshape(B, T, h * w, C)
    avg = jnp.mean(y, axis=2)
    mx = jnp.max(y, axis=2)
    x2 = jnp.concatenate([avg, mx], axis=2)
    mu2 = jnp.mean(x2, axis=-1, keepdims=True)
    v2 = jnp.var(x2, axis=-1, keepdims=True)
    x2 = (x2 - mu2) / jnp.sqrt(v2 + 1e-5) * ln_g + ln_b
    x2 = jax.nn.gelu(x2 @ w1.T + b1, approximate=False)
    Cc = x2.shape[-1]
    local = x2[:, :, :Cc // 2]
    glob = jnp.mean(x2[:, :, Cc // 2:], axis=1, keepdims=True)
    glob = jnp.broadcast_to(glob, (_B, _T, Cc // 2))
    x3 = jnp.concatenate([local, glob], axis=-1)
    s = jax.nn.gelu(x3 @ w2.T + b2, approximate=False)
    s = jax.nn.gelu(s @ w3.T + b3, approximate=False)
    s = (s @ w4.T + b4).squeeze(-1)
    smin = jnp.min(s, axis=-1, keepdims=True)
    smax = jnp.max(s, axis=-1, keepdims=True)
    scores = (s - smin) / (smax - smin + 1e-5)

    noise = jax.random.normal(jax.random.key(42), (_B, _NS, _T), dtype=f32)
    perturbed = scores[:, None, :] + noise * _SIGMA
    pert_t = jnp.swapaxes(perturbed, 1, 2)

    ind = pl.pallas_call(
        _ind_krn,
        grid=(_B,),
        in_specs=[pl.BlockSpec((1, _T, _NS), lambda b: (b, 0, 0))],
        out_specs=pl.BlockSpec((1, _T, _T), lambda b: (b, 0, 0)),
        out_shape=jax.ShapeDtypeStruct((_B, _T, _T), f32),
        interpret=_INTERPRET,
    )(pert_t)

    frames_flat = frames.reshape(_B, _C, _T, _HW)
    tk, bk = pl.pallas_call(
        _apply_krn,
        grid=(_B, _C // _CB),
        in_specs=[
            pl.BlockSpec((1, _CB, _T, _HW), lambda b, c: (b, c, 0, 0)),
            pl.BlockSpec((1, _T, _T), lambda b, c: (b, 0, 0)),
        ],
        out_specs=[
            pl.BlockSpec((1, _CB, _KSEL, _HW), lambda b, c: (b, c, 0, 0)),
            pl.BlockSpec((1, _CB, _T - _KSEL, _HW), lambda b, c: (b, c, 0, 0)),
        ],
        out_shape=[
            jax.ShapeDtypeStruct((_B, _C, _KSEL, _HW), f32),
            jax.ShapeDtypeStruct((_B, _C, _T - _KSEL, _HW), f32),
        ],
        interpret=_INTERPRET,
    )(frames_flat, ind)

    frames_topk = tk.reshape(_B, _C, _KSEL, _H, _W)
    frames_back = bk.reshape(_B, _C, _T - _KSEL, _H, _W)
    sorted_inds = jnp.argsort(scores, axis=1)
    return frames_topk, frames_back, sorted_inds
